# baseline (speedup 1.0000x reference)
import functools

import jax
import jax.numpy as jnp
from jax import lax
from jax.experimental import pallas as pl
from jax.experimental.pallas import tpu as pltpu
from jax.experimental.pallas import tpu_sc as plsc

B, N = 16, 1024
NC, NS, L = 2, 16, 16
NW = NC * NS
RPW = B * N // NW
C = 16

P1, P2, P3 = 0.98745439, -0.4084119, 0.11463969

_mesh = plsc.VectorSubcoreMesh(core_axis_name="c", subcore_axis_name="s")


@functools.partial(
    pl.kernel,
    out_type=[
        jax.ShapeDtypeStruct((NW, L), jnp.float32),
        jax.ShapeDtypeStruct((NW, L), jnp.float32),
    ],
    mesh=_mesh,
    scratch_types=[
        pltpu.VMEM((N,), jnp.float32),
        pltpu.VMEM((C, N), jnp.float32),
        pltpu.VMEM((C, N), jnp.int32),
        pltpu.VMEM((L,), jnp.float32),
        pltpu.SMEM((RPW + C,), jnp.int32),
        pltpu.SemaphoreType.DMA,
    ],
)
def _sc_loss(maskf_hbm, pred_hbm, true_hbm, osum_hbm, ocnt_hbm,
             cm_v, pb_v, tb_v, st_v, ism, sem):
    cid = lax.axis_index("c")
    sid = lax.axis_index("s")
    wid = sid * NC + cid
    b = wid // 2
    h = wid % 2
    row0 = b * N + h * RPW

    pltpu.sync_copy(maskf_hbm.at[b], cm_v)

    iota = lax.iota(jnp.int32, L)

    def cc_body(k, acc):
        return acc + cm_v[pl.ds(k * L, L)]

    ccacc = lax.fori_loop(0, N // L, cc_body, jnp.zeros((L,), jnp.float32))

    def comp_body(k, cnt):
        v = cm_v[pl.ds(h * RPW + k * L, L)]
        for j in range(L):
            ism[cnt] = row0 + k * L + j
            cnt = cnt + jnp.where(v[j] > 0.5, 1, 0)
        return cnt

    cnt = lax.fori_loop(0, RPW // L, comp_body, 0)

    for j in range(C):
        ism[cnt + j] = row0

    ng = (cnt + C - 1) // C

    def g_body(g, vacc):
        idx = jnp.zeros((L,), jnp.int32)
        for j in range(L):
            idx = jnp.where(iota == j, ism[g * C + j], idx)
        cp = pltpu.async_copy(pred_hbm.at[idx], pb_v, sem)
        ct = pltpu.async_copy(true_hbm.at[idx], tb_v, sem)
        cp.wait()
        ct.wait()
        base = g * C
        for r in range(C):
            def col_body(k, racc):
                x = pb_v[r, pl.ds(k * L, L)]
                z = tb_v[r, pl.ds(k * L, L)]
                cf = cm_v[pl.ds(k * L, L)]
                zf = z.astype(jnp.float32)
                a = jnp.abs(x)
                u = jnp.exp(-a)
                p = u * (P1 + u * (P2 + u * P3))
                e = x * (0.5 - zf) + (0.5 * a + p)
                return racc + e * cf

            racc = lax.fori_loop(0, N // L, col_body,
                                 jnp.zeros((L,), jnp.float32))
            w = jnp.where(base + r < cnt, 1.0, 0.0)
            vacc = vacc + racc * w
        return vacc

    vacc = lax.fori_loop(0, ng, g_body, jnp.zeros((L,), jnp.float32))

    st_v[...] = vacc
    pltpu.sync_copy(st_v, osum_hbm.at[wid])
    st_v[...] = ccacc * cnt.astype(jnp.float32)
    pltpu.sync_copy(st_v, ocnt_hbm.at[wid])


def kernel(mask, edge_features_true, edge_features_pred):
    maskf = mask.astype(jnp.float32)
    pred = edge_features_pred.reshape(B * N, N)
    true_ = edge_features_true.astype(jnp.int32).reshape(B * N, N)
    osum, ocnt = _sc_loss(maskf, pred, true_)
    return jnp.sum(osum) / jnp.maximum(jnp.sum(ocnt), 1.0)


# speedup vs baseline: 1.4550x; 1.4550x over previous
import functools

import jax
import jax.numpy as jnp
from jax import lax
from jax.experimental import pallas as pl
from jax.experimental.pallas import tpu as pltpu
from jax.experimental.pallas import tpu_sc as plsc

B, N = 16, 1024
NC, NS, L = 2, 16, 16
NW = NC * NS
RPW = B * N // NW
C = 16
NK = N // L

P1, P2, P3 = 0.98745439, -0.4084119, 0.11463969

_mesh = plsc.VectorSubcoreMesh(core_axis_name="c", subcore_axis_name="s")


@functools.partial(
    pl.kernel,
    out_type=[
        jax.ShapeDtypeStruct((NW, L), jnp.float32),
        jax.ShapeDtypeStruct((NW, L), jnp.float32),
    ],
    mesh=_mesh,
    scratch_types=[
        pltpu.VMEM((N,), jnp.float32),
        pltpu.VMEM((C, N), jnp.float32),
        pltpu.VMEM((C, N), jnp.int32),
        pltpu.VMEM((C, N), jnp.float32),
        pltpu.VMEM((C, N), jnp.int32),
        pltpu.VMEM((4, L), jnp.float32),
        pltpu.VMEM((L,), jnp.float32),
        pltpu.SMEM((RPW + C,), jnp.int32),
        pltpu.SemaphoreType.DMA,
    ],
)
def _sc_loss(maskf_hbm, pred_hbm, true_hbm, osum_hbm, ocnt_hbm,
             cm_v, pa_v, ta_v, pb_v, tb_v, acc_v, st_v, ism, sem):
    cid = lax.axis_index("c")
    sid = lax.axis_index("s")
    wid = sid * NC + cid
    b = wid // 2
    h = wid % 2
    row0 = b * N + h * RPW

    pltpu.sync_copy(maskf_hbm.at[b], cm_v)

    iota = lax.iota(jnp.int32, L)
    zvec = jnp.zeros((L,), jnp.float32)

    def cc_body(k, acc):
        return acc + cm_v[pl.ds(k * L, L)]

    ccacc = lax.fori_loop(0, NK, cc_body, zvec)

    def comp_body(k, cnt):
        v = cm_v[pl.ds(h * RPW + k * L, L)]
        for j in range(L):
            ism[cnt] = row0 + k * L + j
            cnt = cnt + jnp.where(v[j] > 0.5, 1, 0)
        return cnt

    cnt = lax.fori_loop(0, RPW // L, comp_body, 0)

    for j in range(C):
        ism[cnt + j] = row0

    ng = (cnt + C - 1) // C

    for i in range(4):
        acc_v[i] = zvec

    def _build_idx(g):
        idx = jnp.zeros((L,), jnp.int32)
        for j in range(L):
            idx = jnp.where(iota == j, ism[g * C + j], idx)
        return idx

    def _issue(g, p_buf, t_buf):
        idx = _build_idx(g)
        pltpu.async_copy(pred_hbm.at[idx], p_buf, sem)
        pltpu.async_copy(true_hbm.at[idx], t_buf, sem)

    def _wait(p_buf, t_buf):
        pltpu.make_async_copy(pred_hbm.at[pl.ds(0, C)], p_buf, sem).wait()
        pltpu.make_async_copy(true_hbm.at[pl.ds(0, C)], t_buf, sem).wait()

    def _chunk(g, p_buf, t_buf, np_buf, nt_buf):
        _wait(p_buf, t_buf)

        rlo = jnp.clip(cnt - g * C, 0, C)

        def z_body(r, _):
            def zk_body(k, _):
                p_buf[r, pl.ds(k * L, L)] = zvec
                t_buf[r, pl.ds(k * L, L)] = jnp.zeros((L,), jnp.int32)
                return 0
            return lax.fori_loop(0, NK, zk_body, 0)

        lax.fori_loop(rlo, C, z_body, 0)

        @pl.when(g + 1 < ng)
        def _():
            _issue(g + 1, np_buf, nt_buf)

        def col_body(k, accs):
            a0, a1, a2, a3 = accs
            cf = cm_v[pl.ds(k * L, L)]
            for r in range(C):
                x = p_buf[r, pl.ds(k * L, L)]
                z = t_buf[r, pl.ds(k * L, L)]
                zf = z.astype(jnp.float32)
                a = jnp.abs(x)
                u = jnp.exp(-a)
                p = u * (P1 + u * (P2 + u * P3))
                e = x * (0.5 - zf) + (0.5 * a + p)
                t = e * cf
                if r % 4 == 0:
                    a0 = a0 + t
                elif r % 4 == 1:
                    a1 = a1 + t
                elif r % 4 == 2:
                    a2 = a2 + t
                else:
                    a3 = a3 + t
            return (a0, a1, a2, a3)

        accs = lax.fori_loop(0, NK, col_body, (zvec, zvec, zvec, zvec))
        for i in range(4):
            acc_v[i] = acc_v[i] + accs[i]

    @pl.when(ng > 0)
    def _():
        _issue(0, pa_v, ta_v)

    def pair_body(g2, _):
        g = g2 * 2

        @pl.when(g < ng)
        def _():
            _chunk(g, pa_v, ta_v, pb_v, tb_v)

        @pl.when(g + 1 < ng)
        def _():
            _chunk(g + 1, pb_v, tb_v, pa_v, ta_v)

        return 0

    lax.fori_loop(0, (RPW // C + 1) // 2, pair_body, 0)

    vacc = ((acc_v[0] + acc_v[1]) + (acc_v[2] + acc_v[3]))
    psum = 1.0 * (P1 + 1.0 * (P2 + 1.0 * P3))
    npad = (ng * C - cnt).astype(jnp.float32)
    vacc = vacc - ccacc * (npad * psum)

    st_v[...] = vacc
    pltpu.sync_copy(st_v, osum_hbm.at[wid])
    st_v[...] = ccacc * cnt.astype(jnp.float32)
    pltpu.sync_copy(st_v, ocnt_hbm.at[wid])


def kernel(mask, edge_features_true, edge_features_pred):
    maskf = mask.astype(jnp.float32)
    pred = edge_features_pred.reshape(B * N, N)
    true_ = edge_features_true.astype(jnp.int32).reshape(B * N, N)
    osum, ocnt = _sc_loss(maskf, pred, true_)
    return jnp.sum(osum) / jnp.maximum(jnp.sum(ocnt), 1.0)


# speedup vs baseline: 1.7762x; 1.2208x over previous
import functools

import jax
import jax.numpy as jnp
from jax import lax
from jax.experimental import pallas as pl
from jax.experimental.pallas import tpu as pltpu
from jax.experimental.pallas import tpu_sc as plsc

B, N = 16, 1024
NC, NS, L = 2, 16, 16
NW = NC * NS
WPG = NW // B
RPW = N // WPG
C = 16
NK = N // L

Q1, Q2 = 0.94057222, -0.25181309

_mesh = plsc.VectorSubcoreMesh(core_axis_name="c", subcore_axis_name="s")


@functools.partial(
    pl.kernel,
    out_type=[
        jax.ShapeDtypeStruct((NW, L), jnp.float32),
        jax.ShapeDtypeStruct((NW, L), jnp.float32),
    ],
    mesh=_mesh,
    scratch_types=[
        pltpu.VMEM((N,), jnp.float32),
        pltpu.VMEM((C, N), jnp.float32),
        pltpu.VMEM((C, N), jnp.int32),
        pltpu.VMEM((C, N), jnp.float32),
        pltpu.VMEM((C, N), jnp.int32),
        pltpu.VMEM((C, N), jnp.float32),
        pltpu.VMEM((C, N), jnp.int32),
        pltpu.VMEM((RPW // C + 2, L), jnp.int32),
        pltpu.VMEM((1, L), jnp.float32),
        pltpu.VMEM((L,), jnp.float32),
        pltpu.SMEM((RPW + C,), jnp.int32),
        pltpu.SemaphoreType.DMA,
        pltpu.SemaphoreType.DMA,
        pltpu.SemaphoreType.DMA,
    ],
)
def _sc_loss(maskf_hbm, pred_hbm, true_hbm, osum_hbm, ocnt_hbm,
             cm_v, pa_v, ta_v, pb_v, tb_v, pc_v, tc_v, idx_v, acc_v, st_v,
             ism, semA, semB, semC):
    cid = lax.axis_index("c")
    sid = lax.axis_index("s")
    wid = sid * NC + cid
    b = wid // WPG
    h = wid % WPG
    row0 = b * N + h * RPW

    pltpu.sync_copy(maskf_hbm.at[b], cm_v)

    iota = lax.iota(jnp.int32, L)
    zvec = jnp.zeros((L,), jnp.float32)

    def cc_body(k, acc):
        return acc + cm_v[pl.ds(k * L, L)]

    ccacc = lax.fori_loop(0, NK, cc_body, zvec)

    def comp_body(k, cnt):
        v = cm_v[pl.ds(h * RPW + k * L, L)]
        for j in range(L):
            ism[cnt] = row0 + k * L + j
            cnt = cnt + jnp.where(v[j] > 0.5, 1, 0)
        return cnt

    cnt = lax.fori_loop(0, RPW // L, comp_body, 0)

    for j in range(C):
        ism[cnt + j] = row0

    ng = (cnt + C - 1) // C

    acc_v[0] = zvec

    def bi_body(g, _):
        idx = jnp.zeros((L,), jnp.int32)
        for j in range(L):
            idx = jnp.where(iota == j, ism[g * C + j], idx)
        idx_v[g] = idx
        return 0

    lax.fori_loop(0, ng, bi_body, 0)

    def _issue(g, p_buf, t_buf, sem):
        pltpu.async_copy(pred_hbm.at[idx_v.at[g]], p_buf, sem)
        pltpu.async_copy(true_hbm.at[idx_v.at[g]], t_buf, sem)

    def _wait(p_buf, t_buf, sem):
        pltpu.make_async_copy(pred_hbm.at[pl.ds(0, C)], p_buf, sem).wait()
        pltpu.make_async_copy(true_hbm.at[pl.ds(0, C)], t_buf, sem).wait()

    def _chunk(g, p_buf, t_buf, sem, np_buf, nt_buf, nsem):
        @pl.when(g + 2 < ng)
        def _():
            _issue(g + 2, np_buf, nt_buf, nsem)

        _wait(p_buf, t_buf, sem)

        rlo = jnp.clip(cnt - g * C, 0, C)

        def z_body(r, _):
            def zk_body(k, _):
                p_buf[r, pl.ds(k * L, L)] = zvec
                t_buf[r, pl.ds(k * L, L)] = jnp.zeros((L,), jnp.int32)
                return 0
            return lax.fori_loop(0, NK, zk_body, 0)

        lax.fori_loop(rlo, C, z_body, 0)

        def col_body(k, acc):
            cf = cm_v[pl.ds(k * L, L)]
            bs = [zvec, zvec, zvec, zvec]
            for r in range(C):
                x = p_buf[r, pl.ds(k * L, L)]
                z = t_buf[r, pl.ds(k * L, L)]
                xi = lax.bitcast_convert_type(x, jnp.int32)
                y = lax.bitcast_convert_type(xi ^ (z << 31), jnp.float32)
                m = jnp.maximum(y, 0.0)
                nay = lax.bitcast_convert_type(
                    xi | jnp.int32(-2147483648), jnp.float32)
                u = jnp.exp(nay)
                e = m + u * (Q1 + u * Q2)
                bs[r % 4] = bs[r % 4] + e
            srow = (bs[0] + bs[1]) + (bs[2] + bs[3])
            return acc + srow * cf

        accs = lax.fori_loop(0, NK, col_body, zvec)
        acc_v[0] = acc_v[0] + accs

    @pl.when(ng > 0)
    def _():
        _issue(0, pa_v, ta_v, semA)

    @pl.when(ng > 1)
    def _():
        _issue(1, pb_v, tb_v, semB)

    def tri_body(g3, _):
        g = g3 * 3

        @pl.when(g < ng)
        def _():
            _chunk(g, pa_v, ta_v, semA, pc_v, tc_v, semC)

        @pl.when(g + 1 < ng)
        def _():
            _chunk(g + 1, pb_v, tb_v, semB, pa_v, ta_v, semA)

        @pl.when(g + 2 < ng)
        def _():
            _chunk(g + 2, pc_v, tc_v, semC, pb_v, tb_v, semB)

        return 0

    lax.fori_loop(0, (RPW // C + 2) // 3, tri_body, 0)

    vacc = acc_v[0]
    psum = Q1 + Q2
    npad = (ng * C - cnt).astype(jnp.float32)
    vacc = vacc - ccacc * (npad * psum)

    st_v[...] = vacc
    pltpu.sync_copy(st_v, osum_hbm.at[wid])
    st_v[...] = ccacc * cnt.astype(jnp.float32)
    pltpu.sync_copy(st_v, ocnt_hbm.at[wid])


def kernel(mask, edge_features_true, edge_features_pred):
    maskf = mask.astype(jnp.float32)
    pred = edge_features_pred.reshape(B * N, N)
    true_ = edge_features_true.astype(jnp.int32).reshape(B * N, N)
    osum, ocnt = _sc_loss(maskf, pred, true_)
    return jnp.sum(osum) / jnp.maximum(jnp.sum(ocnt), 1.0)
